# baseline (speedup 1.0000x reference)
import functools

import jax
import jax.numpy as jnp
from jax import lax
from jax.experimental import pallas as pl
from jax.experimental.pallas import tpu as pltpu
from jax.experimental.pallas import tpu_sc as plsc


@functools.partial(
    pl.kernel,
    out_type=jax.ShapeDtypeStruct((4, 3, 2, 224, 224), jnp.float32),
    mesh=plsc.VectorSubcoreMesh(core_axis_name="c", subcore_axis_name="s"),
    scratch_types=[
        pltpu.VMEM((1, 224, 224), jnp.float32),
        pltpu.VMEM((16,), jnp.int32),
        pltpu.SemaphoreType.DMA,
    ],
)
def _sc_gather(t_hbm, x_hbm, out_hbm, buf, t_v, sem):
    wid = lax.axis_index("s") * 2 + lax.axis_index("c")

    @pl.when(wid < 24)
    def _():
        pltpu.sync_copy(t_hbm, t_v)
        slot = wid % 2
        bc = wid // 2
        i = bc // 3
        j = bc - i * 3
        vec = t_v[...]
        t = vec[0] * (1 - slot) + vec[1] * slot
        pltpu.async_copy(x_hbm.at[i, j, pl.ds(t, 1)], buf, sem).wait()
        pltpu.sync_copy(buf, out_hbm.at[i, j, pl.ds(slot, 1)])


def kernel(x):
    gap = jax.random.randint(jax.random.key(1), (1,), 2, 16).astype(jnp.int32)
    t_idx = jnp.concatenate(
        [jnp.zeros((1,), dtype=jnp.int32), gap,
         jnp.zeros((14,), dtype=jnp.int32)])
    return _sc_gather(t_idx, x)


# speedup vs baseline: 1.8543x; 1.8543x over previous
import jax
import jax.numpy as jnp
from jax.experimental import pallas as pl
from jax.experimental.pallas import tpu as pltpu

_MIN_GAP = 2
_MAX_GAP = 16


def _copy_body(gap_ref, x_ref, out_ref, buf, sems):
    g = gap_ref[0]
    for h in range(2):
        pltpu.make_async_copy(
            x_ref.at[pl.ds(2 * h, 2), :, pl.ds(0, 1)],
            buf.at[pl.ds(2 * h, 2), :, pl.ds(0, 1)],
            sems.at[2 * h]).start()
        pltpu.make_async_copy(
            x_ref.at[pl.ds(2 * h, 2), :, pl.ds(g, 1)],
            buf.at[pl.ds(2 * h, 2), :, pl.ds(1, 1)],
            sems.at[2 * h + 1]).start()
    for h in range(2):
        pltpu.make_async_copy(
            x_ref.at[pl.ds(2 * h, 2), :, pl.ds(0, 1)],
            buf.at[pl.ds(2 * h, 2), :, pl.ds(0, 1)],
            sems.at[2 * h]).wait()
        pltpu.make_async_copy(
            x_ref.at[pl.ds(2 * h, 2), :, pl.ds(g, 1)],
            buf.at[pl.ds(2 * h, 2), :, pl.ds(1, 1)],
            sems.at[2 * h + 1]).wait()
        pltpu.make_async_copy(
            buf.at[pl.ds(2 * h, 2)], out_ref.at[pl.ds(2 * h, 2)],
            sems.at[4 + h]).start()
    for h in range(2):
        pltpu.make_async_copy(
            buf.at[pl.ds(2 * h, 2)], out_ref.at[pl.ds(2 * h, 2)],
            sems.at[4 + h]).wait()


def kernel(x):
    gap = jax.random.randint(
        jax.random.key(1), (1,), _MIN_GAP, _MAX_GAP).astype(jnp.int32)

    return pl.pallas_call(
        _copy_body,
        out_shape=jax.ShapeDtypeStruct((4, 3, 2, 224, 224), jnp.float32),
        in_specs=[
            pl.BlockSpec(memory_space=pltpu.SMEM),
            pl.BlockSpec(memory_space=pl.ANY),
        ],
        out_specs=pl.BlockSpec(memory_space=pl.ANY),
        scratch_shapes=[
            pltpu.VMEM((4, 3, 2, 224, 224), jnp.float32),
            pltpu.SemaphoreType.DMA((6,)),
        ],
    )(gap, x)
